# Initial kernel scaffold; baseline (speedup 1.0000x reference)
import jax
import jax.numpy as jnp
from jax.experimental import pallas as pl


def kernel(x, weight):
    raise NotImplementedError("write your pallas kernel here")


# speedup vs baseline: 1.0098x; 1.0098x over previous
import functools

import jax
import jax.numpy as jnp
from jax import lax
from jax.experimental import pallas as pl
from jax.experimental.pallas import tpu as pltpu
from jax.experimental.pallas import tpu_sc as plsc

_SCALE = 10.0
_D = 32
_L = 16
_NC = 2
_NS = 16
_NW = _NC * _NS
_CH = 128
_NBUF = 2


@functools.cache
def _build(B: int):
    assert B % (_NW * _CH) == 0
    bpw = B // _NW
    nch = bpw // _CH
    g_steps = nch // _NBUF

    mesh = plsc.VectorSubcoreMesh(core_axis_name="c", subcore_axis_name="s")

    @functools.partial(
        pl.kernel,
        out_type=jax.ShapeDtypeStruct((B, _D), jnp.float32),
        mesh=mesh,
        compiler_params=pltpu.CompilerParams(use_tc_tiling_on_sc=False),
        scratch_types=[
            pltpu.VMEM((nch, _CH), jnp.int32),
            pltpu.VMEM((_NBUF, _CH, _D), jnp.float32),
            pltpu.SemaphoreType.DMA,
            pltpu.SemaphoreType.DMA,
            pltpu.SemaphoreType.DMA,
            pltpu.SemaphoreType.DMA,
        ],
    )
    def scaled_gather(idx_hbm, tbl_hbm, out_hbm, idx_v, rows_v, g0, g1, s0, s1):
        gsem = (g0, g1)
        ssem = (s0, s1)
        wid = lax.axis_index("s") * _NC + lax.axis_index("c")
        cbase = wid * nch

        pltpu.sync_copy(idx_hbm.at[pl.ds(cbase, nch)], idx_v)

        def gather_start(ci_local, b):
            pltpu.async_copy(
                tbl_hbm.at[idx_v.at[ci_local]], rows_v.at[b], gsem[b]
            )

        def gather_wait(ci_local, b):
            pltpu.make_async_copy(
                tbl_hbm.at[idx_v.at[ci_local]], rows_v.at[b], gsem[b]
            ).wait()

        def store_wait(ci_local, b):
            row0 = (cbase + ci_local) * _CH
            pltpu.make_async_copy(
                rows_v.at[b], out_hbm.at[pl.ds(row0, _CH)], ssem[b]
            ).wait()

        def process(ci_local, b):
            gather_wait(ci_local, b)
            for r in range(_CH):
                for h in range(0, _D, _L):
                    rows_v[b, r, pl.ds(h, _L)] = (
                        rows_v[b, r, pl.ds(h, _L)] * _SCALE
                    )
            row0 = (cbase + ci_local) * _CH
            pltpu.async_copy(rows_v.at[b], out_hbm.at[pl.ds(row0, _CH)], ssem[b])

        for b in range(_NBUF):
            gather_start(b, b)

        def step(g, carry):
            for b in range(_NBUF):
                ci = g * _NBUF + b
                process(ci, b)
                store_wait(ci, b)
                gather_start(ci + _NBUF, b)
            return carry

        lax.fori_loop(0, g_steps - 1, step, 0)

        for b in range(_NBUF):
            ci = (g_steps - 1) * _NBUF + b
            process(ci, b)
            store_wait(ci, b)

    return scaled_gather


def kernel(x, weight):
    b0, b1 = x.shape
    flat = b0 * b1
    idx2d = x.reshape(flat // _CH, _CH).astype(jnp.int32)
    out = _build(flat)(idx2d, weight)
    return out.reshape(b0, b1, _D)
